# baseline (speedup 1.0000x reference)
import dataclasses
import functools

import jax
import jax.numpy as jnp
from jax import lax
from jax.experimental import pallas as pl
from jax.experimental.pallas import tpu as pltpu
from jax.experimental.pallas import tpu_sc as plsc

FQ = 5000
FT = 5000
PAD = 5120
K = 32
RB = 256
EPS = 1e-12
NEG = -3.0e38
BIG = 2 ** 30
TPAD = 4.0


def _lex_step(vals, idxs, prev_m, prev_i):
    after = (vals < prev_m) | ((vals == prev_m) & (idxs > prev_i))
    m = jnp.max(jnp.where(after, vals, NEG), axis=1, keepdims=True)
    sel = after & (vals == m)
    ii = jnp.min(jnp.where(sel, idxs, BIG), axis=1, keepdims=True)
    return m, ii


def _score_topk_body(q_ref, t_ref, s_ref, i_ref, p_ref):
    q = q_ref[...]
    t = t_ref[...]
    score = None
    for ax in range(3):
        qmn = jnp.minimum(jnp.minimum(q[:, ax:ax + 1], q[:, ax + 3:ax + 4]),
                          q[:, ax + 6:ax + 7])
        qmx = jnp.maximum(jnp.maximum(q[:, ax:ax + 1], q[:, ax + 3:ax + 4]),
                          q[:, ax + 6:ax + 7])
        tmn = jnp.minimum(jnp.minimum(t[ax:ax + 1, :], t[ax + 3:ax + 4, :]),
                          t[ax + 6:ax + 7, :])
        tmx = jnp.maximum(jnp.maximum(t[ax:ax + 1, :], t[ax + 3:ax + 4, :]),
                          t[ax + 6:ax + 7, :])
        ov = jnp.minimum(qmx, tmx) - jnp.maximum(qmn, tmn)
        score = ov if score is None else jnp.minimum(score, ov)

    iota = lax.broadcasted_iota(jnp.int32, score.shape, 1)
    prev_m = jnp.full((RB, 1), 3.0e38, jnp.float32)
    prev_i = jnp.full((RB, 1), -1, jnp.int32)
    ms, iis = [], []
    for _ in range(K):
        prev_m, prev_i = _lex_step(score, iota, prev_m, prev_i)
        ms.append(prev_m)
        iis.append(prev_i)
    s_ref[...] = jnp.concatenate(ms, axis=1)
    i_ref[...] = jnp.concatenate(iis, axis=1)
    p_ref[...] = jnp.concatenate(
        [(q[:, ax:ax + 1] + q[:, ax + 3:ax + 4] + q[:, ax + 6:ax + 7]) / 3.0
         for ax in range(3)], axis=1)


def _score_topk(q9p, t9T):
    return pl.pallas_call(
        _score_topk_body,
        grid=(PAD // RB,),
        in_specs=[
            pl.BlockSpec((RB, 9), lambda i: (i, 0)),
            pl.BlockSpec((9, PAD), lambda i: (0, 0)),
        ],
        out_specs=[
            pl.BlockSpec((RB, K), lambda i: (i, 0)),
            pl.BlockSpec((RB, K), lambda i: (i, 0)),
            pl.BlockSpec((RB, 3), lambda i: (i, 0)),
        ],
        out_shape=[
            jax.ShapeDtypeStruct((PAD, K), jnp.float32),
            jax.ShapeDtypeStruct((PAD, K), jnp.int32),
            jax.ShapeDtypeStruct((PAD, 3), jnp.float32),
        ],
    )(q9p, t9T)


NW = 32
RW = PAD // NW


def _bary_tail(idx_f, s_f, p_f, t9_f):
    mesh = plsc.VectorSubcoreMesh(core_axis_name="c", subcore_axis_name="s")
    cp = pltpu.CompilerParams()
    if "needs_layout_passes" in pltpu.CompilerParams.__dataclass_fields__:
        cp = dataclasses.replace(cp, needs_layout_passes=False)

    @functools.partial(
        pl.kernel, mesh=mesh, compiler_params=cp,
        out_type=[
            jax.ShapeDtypeStruct((PAD * K,), jnp.int32),
            jax.ShapeDtypeStruct((PAD * K,), jnp.float32),
            jax.ShapeDtypeStruct((PAD * K,), jnp.float32),
            jax.ShapeDtypeStruct((PAD * K,), jnp.float32),
        ],
        scratch_types=[
            pltpu.VMEM((RW * K,), jnp.int32),
            pltpu.VMEM((RW * K,), jnp.float32),
            pltpu.VMEM((RW * 3,), jnp.float32),
            pltpu.VMEM((PAD * 9,), jnp.float32),
            pltpu.VMEM((RW * K,), jnp.int32),
            pltpu.VMEM((RW * K,), jnp.float32),
            pltpu.VMEM((RW * K,), jnp.float32),
            pltpu.VMEM((RW * K,), jnp.float32),
        ],
    )
    def sc_k(idx_hbm, s_hbm, p_hbm, t9_hbm, f_hbm, u_hbm, v_hbm, w_hbm,
             idx_v, s_v, p_v, t9_v, f_v, u_v, v_v, w_v):
        wid = lax.axis_index("s") * 2 + lax.axis_index("c")
        base = wid * (RW * K)
        pbase = wid * (RW * 3)
        pltpu.sync_copy(idx_hbm.at[pl.ds(base, RW * K)], idx_v)
        pltpu.sync_copy(s_hbm.at[pl.ds(base, RW * K)], s_v)
        pltpu.sync_copy(p_hbm.at[pl.ds(pbase, RW * 3)], p_v)
        pltpu.sync_copy(t9_hbm, t9_v)

        zero16 = jnp.zeros((16,), jnp.int32)

        def row(r, carry):
            r3 = r * 3
            px = plsc.load_gather(p_v, [zero16 + r3])
            py = plsc.load_gather(p_v, [zero16 + (r3 + 1)])
            pz = plsc.load_gather(p_v, [zero16 + (r3 + 2)])
            for kk in range(K // 16):
                off = r * K + kk * 16
                sl = pl.ds(off, 16)
                idx16 = idx_v[sl]
                s16 = s_v[sl]
                b9 = idx16 * 9
                g = [plsc.load_gather(t9_v, [b9 + j]) for j in range(9)]
                axx, ayy, azz, bxx, byy, bzz, cxx, cyy, czz = g
                v0x, v0y, v0z = bxx - axx, byy - ayy, bzz - azz
                v1x, v1y, v1z = cxx - axx, cyy - ayy, czz - azz
                v2x, v2y, v2z = px - axx, py - ayy, pz - azz
                d00 = v0x * v0x + v0y * v0y + v0z * v0z
                d01 = v0x * v1x + v0y * v1y + v0z * v1z
                d11 = v1x * v1x + v1y * v1y + v1z * v1z
                d20 = v2x * v0x + v2y * v0y + v2z * v0z
                d21 = v2x * v1x + v2y * v1y + v2z * v1z
                denom = d00 * d11 - d01 * d01 + EPS
                vv = (d11 * d20 - d01 * d21) / denom
                ww = (d00 * d21 - d01 * d20) / denom
                uu = 1.0 - vv - ww
                valid = s16 > 0.0
                zf = jnp.zeros((16,), jnp.float32)
                f_v[sl] = jnp.where(valid, idx16, -1)
                u_v[sl] = jnp.where(valid, uu, zf)
                v_v[sl] = jnp.where(valid, vv, zf)
                w_v[sl] = jnp.where(valid, ww, zf)
            return carry

        lax.fori_loop(0, RW, row, 0)
        pltpu.sync_copy(f_v, f_hbm.at[pl.ds(base, RW * K)])
        pltpu.sync_copy(u_v, u_hbm.at[pl.ds(base, RW * K)])
        pltpu.sync_copy(v_v, v_hbm.at[pl.ds(base, RW * K)])
        pltpu.sync_copy(w_v, w_hbm.at[pl.ds(base, RW * K)])

    return sc_k(idx_f, s_f, p_f, t9_f)


def kernel(query_triangles, target_triangles):
    q9 = query_triangles.reshape(FQ, 9)
    t9 = target_triangles.reshape(FT, 9)
    q9p = jnp.pad(q9, ((0, PAD - FQ), (0, 0)))
    t9p = jnp.pad(t9, ((0, PAD - FT), (0, 0)), constant_values=TPAD)
    t9T = t9p.T

    top_s, top_i, p = _score_topk(q9p, t9T)

    f, u, v, w = _bary_tail(top_i.reshape(-1), top_s.reshape(-1),
                            p.reshape(-1), t9p.reshape(-1))

    faces = f.reshape(PAD, K)[:FQ]
    bcs = jnp.stack([u.reshape(PAD, K)[:FQ],
                     v.reshape(PAD, K)[:FQ],
                     w.reshape(PAD, K)[:FQ]], axis=-1)
    return (faces, bcs)


# speedup vs baseline: 1.0462x; 1.0462x over previous
import dataclasses
import functools

import jax
import jax.numpy as jnp
from jax import lax
from jax.experimental import pallas as pl
from jax.experimental.pallas import tpu as pltpu
from jax.experimental.pallas import tpu_sc as plsc

FQ = 5000
FT = 5000
PAD = 5120
K = 32
RB = 256
EPS = 1e-12
NEG = -3.0e38
BIG = 2 ** 30
TPAD = 4.0


def _lex_step(vals, idxs, prev_m, prev_i):
    ii_tie = jnp.min(jnp.where((vals == prev_m) & (idxs > prev_i), idxs, BIG),
                     axis=1, keepdims=True)
    m2 = jnp.max(jnp.where(vals < prev_m, vals, NEG), axis=1, keepdims=True)
    ii2 = jnp.min(jnp.where(vals == m2, idxs, BIG), axis=1, keepdims=True)
    has_tie = ii_tie < BIG
    m = jnp.where(has_tie, prev_m, m2)
    ii = jnp.where(has_tie, ii_tie, ii2)
    return m, ii


def _score_topk_body(q_ref, t_ref, s_ref, i_ref, p_ref):
    q = q_ref[...]
    t = t_ref[...]
    score = None
    for ax in range(3):
        qmn = jnp.minimum(jnp.minimum(q[:, ax:ax + 1], q[:, ax + 3:ax + 4]),
                          q[:, ax + 6:ax + 7])
        qmx = jnp.maximum(jnp.maximum(q[:, ax:ax + 1], q[:, ax + 3:ax + 4]),
                          q[:, ax + 6:ax + 7])
        tmn = jnp.minimum(jnp.minimum(t[ax:ax + 1, :], t[ax + 3:ax + 4, :]),
                          t[ax + 6:ax + 7, :])
        tmx = jnp.maximum(jnp.maximum(t[ax:ax + 1, :], t[ax + 3:ax + 4, :]),
                          t[ax + 6:ax + 7, :])
        ov = jnp.minimum(qmx, tmx) - jnp.maximum(qmn, tmn)
        score = ov if score is None else jnp.minimum(score, ov)

    iota = lax.broadcasted_iota(jnp.int32, score.shape, 1)
    prev_m = jnp.full((RB, 1), 3.0e38, jnp.float32)
    prev_i = jnp.full((RB, 1), -1, jnp.int32)
    ms, iis = [], []
    for _ in range(K):
        prev_m, prev_i = _lex_step(score, iota, prev_m, prev_i)
        ms.append(prev_m)
        iis.append(prev_i)
    s_ref[...] = jnp.concatenate(ms, axis=1)
    i_ref[...] = jnp.concatenate(iis, axis=1)
    p_ref[...] = jnp.concatenate(
        [(q[:, ax:ax + 1] + q[:, ax + 3:ax + 4] + q[:, ax + 6:ax + 7]) / 3.0
         for ax in range(3)], axis=1)


def _score_topk(q9p, t9T):
    return pl.pallas_call(
        _score_topk_body,
        grid=(PAD // RB,),
        in_specs=[
            pl.BlockSpec((RB, 9), lambda i: (i, 0)),
            pl.BlockSpec((9, PAD), lambda i: (0, 0)),
        ],
        out_specs=[
            pl.BlockSpec((RB, K), lambda i: (i, 0)),
            pl.BlockSpec((RB, K), lambda i: (i, 0)),
            pl.BlockSpec((RB, 3), lambda i: (i, 0)),
        ],
        out_shape=[
            jax.ShapeDtypeStruct((PAD, K), jnp.float32),
            jax.ShapeDtypeStruct((PAD, K), jnp.int32),
            jax.ShapeDtypeStruct((PAD, 3), jnp.float32),
        ],
    )(q9p, t9T)


NW = 32
RW = PAD // NW


def _bary_tail(idx_f, s_f, p_f, t9_f):
    mesh = plsc.VectorSubcoreMesh(core_axis_name="c", subcore_axis_name="s")
    cp = pltpu.CompilerParams()
    if "needs_layout_passes" in pltpu.CompilerParams.__dataclass_fields__:
        cp = dataclasses.replace(cp, needs_layout_passes=False)

    @functools.partial(
        pl.kernel, mesh=mesh, compiler_params=cp,
        out_type=[
            jax.ShapeDtypeStruct((PAD * K,), jnp.int32),
            jax.ShapeDtypeStruct((PAD * K,), jnp.float32),
            jax.ShapeDtypeStruct((PAD * K,), jnp.float32),
            jax.ShapeDtypeStruct((PAD * K,), jnp.float32),
        ],
        scratch_types=[
            pltpu.VMEM((RW * K,), jnp.int32),
            pltpu.VMEM((RW * K,), jnp.float32),
            pltpu.VMEM((RW * 3,), jnp.float32),
            pltpu.VMEM((PAD * 9,), jnp.float32),
            pltpu.VMEM((RW * K,), jnp.int32),
            pltpu.VMEM((RW * K,), jnp.float32),
            pltpu.VMEM((RW * K,), jnp.float32),
            pltpu.VMEM((RW * K,), jnp.float32),
        ],
    )
    def sc_k(idx_hbm, s_hbm, p_hbm, t9_hbm, f_hbm, u_hbm, v_hbm, w_hbm,
             idx_v, s_v, p_v, t9_v, f_v, u_v, v_v, w_v):
        wid = lax.axis_index("s") * 2 + lax.axis_index("c")
        base = wid * (RW * K)
        pbase = wid * (RW * 3)
        pltpu.sync_copy(idx_hbm.at[pl.ds(base, RW * K)], idx_v)
        pltpu.sync_copy(s_hbm.at[pl.ds(base, RW * K)], s_v)
        pltpu.sync_copy(p_hbm.at[pl.ds(pbase, RW * 3)], p_v)
        pltpu.sync_copy(t9_hbm, t9_v)

        zero16 = jnp.zeros((16,), jnp.int32)

        def row(r, carry):
            r3 = r * 3
            px = plsc.load_gather(p_v, [zero16 + r3])
            py = plsc.load_gather(p_v, [zero16 + (r3 + 1)])
            pz = plsc.load_gather(p_v, [zero16 + (r3 + 2)])
            for kk in range(K // 16):
                off = r * K + kk * 16
                sl = pl.ds(off, 16)
                idx16 = idx_v[sl]
                s16 = s_v[sl]
                b9 = idx16 * 9
                g = [plsc.load_gather(t9_v, [b9 + j]) for j in range(9)]
                axx, ayy, azz, bxx, byy, bzz, cxx, cyy, czz = g
                v0x, v0y, v0z = bxx - axx, byy - ayy, bzz - azz
                v1x, v1y, v1z = cxx - axx, cyy - ayy, czz - azz
                v2x, v2y, v2z = px - axx, py - ayy, pz - azz
                d00 = v0x * v0x + v0y * v0y + v0z * v0z
                d01 = v0x * v1x + v0y * v1y + v0z * v1z
                d11 = v1x * v1x + v1y * v1y + v1z * v1z
                d20 = v2x * v0x + v2y * v0y + v2z * v0z
                d21 = v2x * v1x + v2y * v1y + v2z * v1z
                denom = d00 * d11 - d01 * d01 + EPS
                vv = (d11 * d20 - d01 * d21) / denom
                ww = (d00 * d21 - d01 * d20) / denom
                uu = 1.0 - vv - ww
                valid = s16 > 0.0
                zf = jnp.zeros((16,), jnp.float32)
                f_v[sl] = jnp.where(valid, idx16, -1)
                u_v[sl] = jnp.where(valid, uu, zf)
                v_v[sl] = jnp.where(valid, vv, zf)
                w_v[sl] = jnp.where(valid, ww, zf)
            return carry

        lax.fori_loop(0, RW, row, 0)
        pltpu.sync_copy(f_v, f_hbm.at[pl.ds(base, RW * K)])
        pltpu.sync_copy(u_v, u_hbm.at[pl.ds(base, RW * K)])
        pltpu.sync_copy(v_v, v_hbm.at[pl.ds(base, RW * K)])
        pltpu.sync_copy(w_v, w_hbm.at[pl.ds(base, RW * K)])

    return sc_k(idx_f, s_f, p_f, t9_f)


def kernel(query_triangles, target_triangles):
    q9 = query_triangles.reshape(FQ, 9)
    t9 = target_triangles.reshape(FT, 9)
    q9p = jnp.pad(q9, ((0, PAD - FQ), (0, 0)))
    t9p = jnp.pad(t9, ((0, PAD - FT), (0, 0)), constant_values=TPAD)
    t9T = t9p.T

    top_s, top_i, p = _score_topk(q9p, t9T)

    f, u, v, w = _bary_tail(top_i.reshape(-1), top_s.reshape(-1),
                            p.reshape(-1), t9p.reshape(-1))

    faces = f.reshape(PAD, K)[:FQ]
    bcs = jnp.stack([u.reshape(PAD, K)[:FQ],
                     v.reshape(PAD, K)[:FQ],
                     w.reshape(PAD, K)[:FQ]], axis=-1)
    return (faces, bcs)


# speedup vs baseline: 1.1548x; 1.1038x over previous
import dataclasses
import functools

import jax
import jax.numpy as jnp
from jax import lax
from jax.experimental import pallas as pl
from jax.experimental.pallas import tpu as pltpu
from jax.experimental.pallas import tpu_sc as plsc

FQ = 5000
FT = 5000
PAD = 5120
K = 32
RB = 512
EPS = 1e-12
NEG = -3.0e38
BIG = 2 ** 30
TPAD = 4.0


def _lex_step(vals, idxs, prev_m, prev_i):
    elig = (vals < prev_m) | ((vals == prev_m) & (idxs > prev_i))
    masked = jnp.where(elig, vals, NEG)
    m = jnp.max(masked, axis=1, keepdims=True)
    ii = jnp.min(jnp.where(masked == m, idxs, BIG), axis=1, keepdims=True)
    return m, ii


def _score_topk_body(q_ref, t_ref, s_ref, i_ref, p_ref):
    q = q_ref[...]
    t = t_ref[...]
    score = None
    for ax in range(3):
        qmn = jnp.minimum(jnp.minimum(q[:, ax:ax + 1], q[:, ax + 3:ax + 4]),
                          q[:, ax + 6:ax + 7])
        qmx = jnp.maximum(jnp.maximum(q[:, ax:ax + 1], q[:, ax + 3:ax + 4]),
                          q[:, ax + 6:ax + 7])
        tmn = jnp.minimum(jnp.minimum(t[ax:ax + 1, :], t[ax + 3:ax + 4, :]),
                          t[ax + 6:ax + 7, :])
        tmx = jnp.maximum(jnp.maximum(t[ax:ax + 1, :], t[ax + 3:ax + 4, :]),
                          t[ax + 6:ax + 7, :])
        ov = jnp.minimum(qmx, tmx) - jnp.maximum(qmn, tmn)
        score = ov if score is None else jnp.minimum(score, ov)

    iota = lax.broadcasted_iota(jnp.int32, score.shape, 1)
    prev_m = jnp.full((RB, 1), 3.0e38, jnp.float32)
    prev_i = jnp.full((RB, 1), -1, jnp.int32)
    ms, iis = [], []
    for _ in range(K):
        prev_m, prev_i = _lex_step(score, iota, prev_m, prev_i)
        ms.append(prev_m)
        iis.append(prev_i)
    s_ref[...] = jnp.concatenate(ms, axis=1)
    i_ref[...] = jnp.concatenate(iis, axis=1)
    p_ref[...] = jnp.concatenate(
        [(q[:, ax:ax + 1] + q[:, ax + 3:ax + 4] + q[:, ax + 6:ax + 7]) / 3.0
         for ax in range(3)], axis=1)


def _score_topk(q9p, t9T):
    return pl.pallas_call(
        _score_topk_body,
        grid=(PAD // RB,),
        in_specs=[
            pl.BlockSpec((RB, 9), lambda i: (i, 0)),
            pl.BlockSpec((9, PAD), lambda i: (0, 0)),
        ],
        out_specs=[
            pl.BlockSpec((RB, K), lambda i: (i, 0)),
            pl.BlockSpec((RB, K), lambda i: (i, 0)),
            pl.BlockSpec((RB, 3), lambda i: (i, 0)),
        ],
        out_shape=[
            jax.ShapeDtypeStruct((PAD, K), jnp.float32),
            jax.ShapeDtypeStruct((PAD, K), jnp.int32),
            jax.ShapeDtypeStruct((PAD, 3), jnp.float32),
        ],
    )(q9p, t9T)


NW = 32
RW = PAD // NW


def _bary_tail(idx_f, s_f, p_f, t9_f):
    mesh = plsc.VectorSubcoreMesh(core_axis_name="c", subcore_axis_name="s")
    cp = pltpu.CompilerParams()
    if "needs_layout_passes" in pltpu.CompilerParams.__dataclass_fields__:
        cp = dataclasses.replace(cp, needs_layout_passes=False)

    @functools.partial(
        pl.kernel, mesh=mesh, compiler_params=cp,
        out_type=[
            jax.ShapeDtypeStruct((PAD * K,), jnp.int32),
            jax.ShapeDtypeStruct((PAD * K,), jnp.float32),
            jax.ShapeDtypeStruct((PAD * K,), jnp.float32),
            jax.ShapeDtypeStruct((PAD * K,), jnp.float32),
        ],
        scratch_types=[
            pltpu.VMEM((RW * K,), jnp.int32),
            pltpu.VMEM((RW * K,), jnp.float32),
            pltpu.VMEM((RW * 3,), jnp.float32),
            pltpu.VMEM((PAD * 9,), jnp.float32),
            pltpu.VMEM((RW * K,), jnp.int32),
            pltpu.VMEM((RW * K,), jnp.float32),
            pltpu.VMEM((RW * K,), jnp.float32),
            pltpu.VMEM((RW * K,), jnp.float32),
        ],
    )
    def sc_k(idx_hbm, s_hbm, p_hbm, t9_hbm, f_hbm, u_hbm, v_hbm, w_hbm,
             idx_v, s_v, p_v, t9_v, f_v, u_v, v_v, w_v):
        wid = lax.axis_index("s") * 2 + lax.axis_index("c")
        base = wid * (RW * K)
        pbase = wid * (RW * 3)
        pltpu.sync_copy(idx_hbm.at[pl.ds(base, RW * K)], idx_v)
        pltpu.sync_copy(s_hbm.at[pl.ds(base, RW * K)], s_v)
        pltpu.sync_copy(p_hbm.at[pl.ds(pbase, RW * 3)], p_v)
        pltpu.sync_copy(t9_hbm, t9_v)

        zero16 = jnp.zeros((16,), jnp.int32)

        def row(r, carry):
            r3 = r * 3
            px = plsc.load_gather(p_v, [zero16 + r3])
            py = plsc.load_gather(p_v, [zero16 + (r3 + 1)])
            pz = plsc.load_gather(p_v, [zero16 + (r3 + 2)])
            for kk in range(K // 16):
                off = r * K + kk * 16
                sl = pl.ds(off, 16)
                idx16 = idx_v[sl]
                s16 = s_v[sl]
                b9 = idx16 * 9
                g = [plsc.load_gather(t9_v, [b9 + j]) for j in range(9)]
                axx, ayy, azz, bxx, byy, bzz, cxx, cyy, czz = g
                v0x, v0y, v0z = bxx - axx, byy - ayy, bzz - azz
                v1x, v1y, v1z = cxx - axx, cyy - ayy, czz - azz
                v2x, v2y, v2z = px - axx, py - ayy, pz - azz
                d00 = v0x * v0x + v0y * v0y + v0z * v0z
                d01 = v0x * v1x + v0y * v1y + v0z * v1z
                d11 = v1x * v1x + v1y * v1y + v1z * v1z
                d20 = v2x * v0x + v2y * v0y + v2z * v0z
                d21 = v2x * v1x + v2y * v1y + v2z * v1z
                denom = d00 * d11 - d01 * d01 + EPS
                vv = (d11 * d20 - d01 * d21) / denom
                ww = (d00 * d21 - d01 * d20) / denom
                uu = 1.0 - vv - ww
                valid = s16 > 0.0
                zf = jnp.zeros((16,), jnp.float32)
                f_v[sl] = jnp.where(valid, idx16, -1)
                u_v[sl] = jnp.where(valid, uu, zf)
                v_v[sl] = jnp.where(valid, vv, zf)
                w_v[sl] = jnp.where(valid, ww, zf)
            return carry

        lax.fori_loop(0, RW, row, 0)
        pltpu.sync_copy(f_v, f_hbm.at[pl.ds(base, RW * K)])
        pltpu.sync_copy(u_v, u_hbm.at[pl.ds(base, RW * K)])
        pltpu.sync_copy(v_v, v_hbm.at[pl.ds(base, RW * K)])
        pltpu.sync_copy(w_v, w_hbm.at[pl.ds(base, RW * K)])

    return sc_k(idx_f, s_f, p_f, t9_f)


def kernel(query_triangles, target_triangles):
    q9 = query_triangles.reshape(FQ, 9)
    t9 = target_triangles.reshape(FT, 9)
    q9p = jnp.pad(q9, ((0, PAD - FQ), (0, 0)))
    t9p = jnp.pad(t9, ((0, PAD - FT), (0, 0)), constant_values=TPAD)
    t9T = t9p.T

    top_s, top_i, p = _score_topk(q9p, t9T)

    f, u, v, w = _bary_tail(top_i.reshape(-1), top_s.reshape(-1),
                            p.reshape(-1), t9p.reshape(-1))

    faces = f.reshape(PAD, K)[:FQ]
    bcs = jnp.stack([u.reshape(PAD, K)[:FQ],
                     v.reshape(PAD, K)[:FQ],
                     w.reshape(PAD, K)[:FQ]], axis=-1)
    return (faces, bcs)


# speedup vs baseline: 1.1838x; 1.0251x over previous
import dataclasses
import functools

import jax
import jax.numpy as jnp
from jax import lax
from jax.experimental import pallas as pl
from jax.experimental.pallas import tpu as pltpu
from jax.experimental.pallas import tpu_sc as plsc

FQ = 5000
FT = 5000
PAD = 5120
K = 32
RB = 640
EPS = 1e-12
NEG = -3.0e38
BIG = 2 ** 30
TPAD = 4.0


def _lex_step(vals, idxs, prev_m, prev_i):
    elig = (vals < prev_m) | ((vals == prev_m) & (idxs > prev_i))
    masked = jnp.where(elig, vals, NEG)
    m = jnp.max(masked, axis=1, keepdims=True)
    ii = jnp.min(jnp.where(masked == m, idxs, BIG), axis=1, keepdims=True)
    return m, ii


def _score_topk_body(q_ref, t_ref, s_ref, i_ref, p_ref):
    q = q_ref[...]
    t = t_ref[...]
    score = None
    for ax in range(3):
        qmn = jnp.minimum(jnp.minimum(q[:, ax:ax + 1], q[:, ax + 3:ax + 4]),
                          q[:, ax + 6:ax + 7])
        qmx = jnp.maximum(jnp.maximum(q[:, ax:ax + 1], q[:, ax + 3:ax + 4]),
                          q[:, ax + 6:ax + 7])
        tmn = jnp.minimum(jnp.minimum(t[ax:ax + 1, :], t[ax + 3:ax + 4, :]),
                          t[ax + 6:ax + 7, :])
        tmx = jnp.maximum(jnp.maximum(t[ax:ax + 1, :], t[ax + 3:ax + 4, :]),
                          t[ax + 6:ax + 7, :])
        ov = jnp.minimum(qmx, tmx) - jnp.maximum(qmn, tmn)
        score = ov if score is None else jnp.minimum(score, ov)

    iota = lax.broadcasted_iota(jnp.int32, score.shape, 1)
    prev_m = jnp.full((RB, 1), 3.0e38, jnp.float32)
    prev_i = jnp.full((RB, 1), -1, jnp.int32)
    ms, iis = [], []
    for _ in range(K):
        prev_m, prev_i = _lex_step(score, iota, prev_m, prev_i)
        ms.append(prev_m)
        iis.append(prev_i)
    s_ref[...] = jnp.concatenate(ms, axis=1)
    i_ref[...] = jnp.concatenate(iis, axis=1)
    p_ref[...] = jnp.concatenate(
        [(q[:, ax:ax + 1] + q[:, ax + 3:ax + 4] + q[:, ax + 6:ax + 7]) / 3.0
         for ax in range(3)], axis=1)


def _score_topk(q9p, t9T):
    return pl.pallas_call(
        _score_topk_body,
        grid=(PAD // RB,),
        in_specs=[
            pl.BlockSpec((RB, 9), lambda i: (i, 0)),
            pl.BlockSpec((9, PAD), lambda i: (0, 0)),
        ],
        out_specs=[
            pl.BlockSpec((RB, K), lambda i: (i, 0)),
            pl.BlockSpec((RB, K), lambda i: (i, 0)),
            pl.BlockSpec((RB, 3), lambda i: (i, 0)),
        ],
        out_shape=[
            jax.ShapeDtypeStruct((PAD, K), jnp.float32),
            jax.ShapeDtypeStruct((PAD, K), jnp.int32),
            jax.ShapeDtypeStruct((PAD, 3), jnp.float32),
        ],
    )(q9p, t9T)


NW = 32
RW = PAD // NW


def _bary_tail(idx_f, s_f, p_f, t9_f):
    mesh = plsc.VectorSubcoreMesh(core_axis_name="c", subcore_axis_name="s")
    cp = pltpu.CompilerParams()
    if "needs_layout_passes" in pltpu.CompilerParams.__dataclass_fields__:
        cp = dataclasses.replace(cp, needs_layout_passes=False)

    @functools.partial(
        pl.kernel, mesh=mesh, compiler_params=cp,
        out_type=[
            jax.ShapeDtypeStruct((PAD * K,), jnp.int32),
            jax.ShapeDtypeStruct((PAD * K,), jnp.float32),
            jax.ShapeDtypeStruct((PAD * K,), jnp.float32),
            jax.ShapeDtypeStruct((PAD * K,), jnp.float32),
        ],
        scratch_types=[
            pltpu.VMEM((RW * K,), jnp.int32),
            pltpu.VMEM((RW * K,), jnp.float32),
            pltpu.VMEM((RW * 3,), jnp.float32),
            pltpu.VMEM((PAD * 9,), jnp.float32),
            pltpu.VMEM((RW * K,), jnp.int32),
            pltpu.VMEM((RW * K,), jnp.float32),
            pltpu.VMEM((RW * K,), jnp.float32),
            pltpu.VMEM((RW * K,), jnp.float32),
        ],
    )
    def sc_k(idx_hbm, s_hbm, p_hbm, t9_hbm, f_hbm, u_hbm, v_hbm, w_hbm,
             idx_v, s_v, p_v, t9_v, f_v, u_v, v_v, w_v):
        wid = lax.axis_index("s") * 2 + lax.axis_index("c")
        base = wid * (RW * K)
        pbase = wid * (RW * 3)
        pltpu.sync_copy(idx_hbm.at[pl.ds(base, RW * K)], idx_v)
        pltpu.sync_copy(s_hbm.at[pl.ds(base, RW * K)], s_v)
        pltpu.sync_copy(p_hbm.at[pl.ds(pbase, RW * 3)], p_v)
        pltpu.sync_copy(t9_hbm, t9_v)

        zero16 = jnp.zeros((16,), jnp.int32)

        def row(r, carry):
            r3 = r * 3
            px = plsc.load_gather(p_v, [zero16 + r3])
            py = plsc.load_gather(p_v, [zero16 + (r3 + 1)])
            pz = plsc.load_gather(p_v, [zero16 + (r3 + 2)])
            for kk in range(K // 16):
                off = r * K + kk * 16
                sl = pl.ds(off, 16)
                idx16 = idx_v[sl]
                s16 = s_v[sl]
                b9 = idx16 * 9
                g = [plsc.load_gather(t9_v, [b9 + j]) for j in range(9)]
                axx, ayy, azz, bxx, byy, bzz, cxx, cyy, czz = g
                v0x, v0y, v0z = bxx - axx, byy - ayy, bzz - azz
                v1x, v1y, v1z = cxx - axx, cyy - ayy, czz - azz
                v2x, v2y, v2z = px - axx, py - ayy, pz - azz
                d00 = v0x * v0x + v0y * v0y + v0z * v0z
                d01 = v0x * v1x + v0y * v1y + v0z * v1z
                d11 = v1x * v1x + v1y * v1y + v1z * v1z
                d20 = v2x * v0x + v2y * v0y + v2z * v0z
                d21 = v2x * v1x + v2y * v1y + v2z * v1z
                denom = d00 * d11 - d01 * d01 + EPS
                vv = (d11 * d20 - d01 * d21) / denom
                ww = (d00 * d21 - d01 * d20) / denom
                uu = 1.0 - vv - ww
                valid = s16 > 0.0
                zf = jnp.zeros((16,), jnp.float32)
                f_v[sl] = jnp.where(valid, idx16, -1)
                u_v[sl] = jnp.where(valid, uu, zf)
                v_v[sl] = jnp.where(valid, vv, zf)
                w_v[sl] = jnp.where(valid, ww, zf)
            return carry

        lax.fori_loop(0, RW, row, 0)
        pltpu.sync_copy(f_v, f_hbm.at[pl.ds(base, RW * K)])
        pltpu.sync_copy(u_v, u_hbm.at[pl.ds(base, RW * K)])
        pltpu.sync_copy(v_v, v_hbm.at[pl.ds(base, RW * K)])
        pltpu.sync_copy(w_v, w_hbm.at[pl.ds(base, RW * K)])

    return sc_k(idx_f, s_f, p_f, t9_f)


def kernel(query_triangles, target_triangles):
    q9 = query_triangles.reshape(FQ, 9)
    t9 = target_triangles.reshape(FT, 9)
    q9p = jnp.pad(q9, ((0, PAD - FQ), (0, 0)))
    t9p = jnp.pad(t9, ((0, PAD - FT), (0, 0)), constant_values=TPAD)
    t9T = t9p.T

    top_s, top_i, p = _score_topk(q9p, t9T)

    f, u, v, w = _bary_tail(top_i.reshape(-1), top_s.reshape(-1),
                            p.reshape(-1), t9p.reshape(-1))

    faces = f.reshape(PAD, K)[:FQ]
    bcs = jnp.stack([u.reshape(PAD, K)[:FQ],
                     v.reshape(PAD, K)[:FQ],
                     w.reshape(PAD, K)[:FQ]], axis=-1)
    return (faces, bcs)


# speedup vs baseline: 1.7919x; 1.5137x over previous
import dataclasses
import functools

import jax
import jax.numpy as jnp
from jax import lax
from jax.experimental import pallas as pl
from jax.experimental.pallas import tpu as pltpu
from jax.experimental.pallas import tpu_sc as plsc

FQ = 5000
FT = 5000
PAD = 5120
K = 32
RB = 640
EPS = 1e-12
NEG = -3.0e38
BIG = 2 ** 30
TPAD = 4.0


def _lex_step(vals, idxs, prev_m, prev_i):
    elig = (vals < prev_m) | ((vals == prev_m) & (idxs > prev_i))
    masked = jnp.where(elig, vals, NEG)
    m = jnp.max(masked, axis=1, keepdims=True)
    ii = jnp.min(jnp.where(masked == m, idxs, BIG), axis=1, keepdims=True)
    return m, ii


def _score_topk_body(q_ref, t_ref, s_ref, i_ref, p_ref):
    q = q_ref[...]
    t = t_ref[...]
    score = None
    for ax in range(3):
        qmn = jnp.minimum(jnp.minimum(q[:, ax:ax + 1], q[:, ax + 3:ax + 4]),
                          q[:, ax + 6:ax + 7])
        qmx = jnp.maximum(jnp.maximum(q[:, ax:ax + 1], q[:, ax + 3:ax + 4]),
                          q[:, ax + 6:ax + 7])
        tmn = jnp.minimum(jnp.minimum(t[ax:ax + 1, :], t[ax + 3:ax + 4, :]),
                          t[ax + 6:ax + 7, :])
        tmx = jnp.maximum(jnp.maximum(t[ax:ax + 1, :], t[ax + 3:ax + 4, :]),
                          t[ax + 6:ax + 7, :])
        ov = jnp.minimum(qmx, tmx) - jnp.maximum(qmn, tmn)
        score = ov if score is None else jnp.minimum(score, ov)

    iota = lax.broadcasted_iota(jnp.int32, score.shape, 1)
    masked = score
    ms, iis = [], []
    for k in range(K):
        m = jnp.max(masked, axis=1, keepdims=True)
        ii = jnp.min(jnp.where(masked == m, iota, BIG), axis=1, keepdims=True)
        ms.append(m)
        iis.append(ii)
        if k + 1 < K:
            masked = jnp.where(iota == ii, NEG, masked)
    s_ref[...] = jnp.concatenate(ms, axis=1)
    i_ref[...] = jnp.concatenate(iis, axis=1)
    p_ref[...] = jnp.concatenate(
        [(q[:, ax:ax + 1] + q[:, ax + 3:ax + 4] + q[:, ax + 6:ax + 7]) / 3.0
         for ax in range(3)], axis=1)


def _score_topk(q9p, t9T):
    return pl.pallas_call(
        _score_topk_body,
        grid=(PAD // RB,),
        in_specs=[
            pl.BlockSpec((RB, 9), lambda i: (i, 0)),
            pl.BlockSpec((9, PAD), lambda i: (0, 0)),
        ],
        out_specs=[
            pl.BlockSpec((RB, K), lambda i: (i, 0)),
            pl.BlockSpec((RB, K), lambda i: (i, 0)),
            pl.BlockSpec((RB, 3), lambda i: (i, 0)),
        ],
        out_shape=[
            jax.ShapeDtypeStruct((PAD, K), jnp.float32),
            jax.ShapeDtypeStruct((PAD, K), jnp.int32),
            jax.ShapeDtypeStruct((PAD, 3), jnp.float32),
        ],
    )(q9p, t9T)


NW = 32
RW = PAD // NW


def _bary_tail(idx_f, s_f, p_f, t9_f):
    mesh = plsc.VectorSubcoreMesh(core_axis_name="c", subcore_axis_name="s")
    cp = pltpu.CompilerParams()
    if "needs_layout_passes" in pltpu.CompilerParams.__dataclass_fields__:
        cp = dataclasses.replace(cp, needs_layout_passes=False)

    @functools.partial(
        pl.kernel, mesh=mesh, compiler_params=cp,
        out_type=[
            jax.ShapeDtypeStruct((PAD * K,), jnp.int32),
            jax.ShapeDtypeStruct((PAD * K,), jnp.float32),
            jax.ShapeDtypeStruct((PAD * K,), jnp.float32),
            jax.ShapeDtypeStruct((PAD * K,), jnp.float32),
        ],
        scratch_types=[
            pltpu.VMEM((RW * K,), jnp.int32),
            pltpu.VMEM((RW * K,), jnp.float32),
            pltpu.VMEM((RW * 3,), jnp.float32),
            pltpu.VMEM((PAD * 9,), jnp.float32),
            pltpu.VMEM((RW * K,), jnp.int32),
            pltpu.VMEM((RW * K,), jnp.float32),
            pltpu.VMEM((RW * K,), jnp.float32),
            pltpu.VMEM((RW * K,), jnp.float32),
        ],
    )
    def sc_k(idx_hbm, s_hbm, p_hbm, t9_hbm, f_hbm, u_hbm, v_hbm, w_hbm,
             idx_v, s_v, p_v, t9_v, f_v, u_v, v_v, w_v):
        wid = lax.axis_index("s") * 2 + lax.axis_index("c")
        base = wid * (RW * K)
        pbase = wid * (RW * 3)
        pltpu.sync_copy(idx_hbm.at[pl.ds(base, RW * K)], idx_v)
        pltpu.sync_copy(s_hbm.at[pl.ds(base, RW * K)], s_v)
        pltpu.sync_copy(p_hbm.at[pl.ds(pbase, RW * 3)], p_v)
        pltpu.sync_copy(t9_hbm, t9_v)

        zero16 = jnp.zeros((16,), jnp.int32)

        def row(r, carry):
            r3 = r * 3
            px = plsc.load_gather(p_v, [zero16 + r3])
            py = plsc.load_gather(p_v, [zero16 + (r3 + 1)])
            pz = plsc.load_gather(p_v, [zero16 + (r3 + 2)])
            for kk in range(K // 16):
                off = r * K + kk * 16
                sl = pl.ds(off, 16)
                idx16 = idx_v[sl]
                s16 = s_v[sl]
                b9 = idx16 * 9
                g = [plsc.load_gather(t9_v, [b9 + j]) for j in range(9)]
                axx, ayy, azz, bxx, byy, bzz, cxx, cyy, czz = g
                v0x, v0y, v0z = bxx - axx, byy - ayy, bzz - azz
                v1x, v1y, v1z = cxx - axx, cyy - ayy, czz - azz
                v2x, v2y, v2z = px - axx, py - ayy, pz - azz
                d00 = v0x * v0x + v0y * v0y + v0z * v0z
                d01 = v0x * v1x + v0y * v1y + v0z * v1z
                d11 = v1x * v1x + v1y * v1y + v1z * v1z
                d20 = v2x * v0x + v2y * v0y + v2z * v0z
                d21 = v2x * v1x + v2y * v1y + v2z * v1z
                denom = d00 * d11 - d01 * d01 + EPS
                vv = (d11 * d20 - d01 * d21) / denom
                ww = (d00 * d21 - d01 * d20) / denom
                uu = 1.0 - vv - ww
                valid = s16 > 0.0
                zf = jnp.zeros((16,), jnp.float32)
                f_v[sl] = jnp.where(valid, idx16, -1)
                u_v[sl] = jnp.where(valid, uu, zf)
                v_v[sl] = jnp.where(valid, vv, zf)
                w_v[sl] = jnp.where(valid, ww, zf)
            return carry

        lax.fori_loop(0, RW, row, 0)
        pltpu.sync_copy(f_v, f_hbm.at[pl.ds(base, RW * K)])
        pltpu.sync_copy(u_v, u_hbm.at[pl.ds(base, RW * K)])
        pltpu.sync_copy(v_v, v_hbm.at[pl.ds(base, RW * K)])
        pltpu.sync_copy(w_v, w_hbm.at[pl.ds(base, RW * K)])

    return sc_k(idx_f, s_f, p_f, t9_f)


def kernel(query_triangles, target_triangles):
    q9 = query_triangles.reshape(FQ, 9)
    t9 = target_triangles.reshape(FT, 9)
    q9p = jnp.pad(q9, ((0, PAD - FQ), (0, 0)))
    t9p = jnp.pad(t9, ((0, PAD - FT), (0, 0)), constant_values=TPAD)
    t9T = t9p.T

    top_s, top_i, p = _score_topk(q9p, t9T)

    f, u, v, w = _bary_tail(top_i.reshape(-1), top_s.reshape(-1),
                            p.reshape(-1), t9p.reshape(-1))

    faces = f.reshape(PAD, K)[:FQ]
    bcs = jnp.stack([u.reshape(PAD, K)[:FQ],
                     v.reshape(PAD, K)[:FQ],
                     w.reshape(PAD, K)[:FQ]], axis=-1)
    return (faces, bcs)


# speedup vs baseline: 1.8403x; 1.0270x over previous
import dataclasses
import functools

import jax
import jax.numpy as jnp
from jax import lax
from jax.experimental import pallas as pl
from jax.experimental.pallas import tpu as pltpu
from jax.experimental.pallas import tpu_sc as plsc

FQ = 5000
FT = 5000
PAD = 5120
K = 32
RB = 1024
EPS = 1e-12
NEG = -3.0e38
BIG = 2 ** 30
TPAD = 4.0


def _lex_step(vals, idxs, prev_m, prev_i):
    elig = (vals < prev_m) | ((vals == prev_m) & (idxs > prev_i))
    masked = jnp.where(elig, vals, NEG)
    m = jnp.max(masked, axis=1, keepdims=True)
    ii = jnp.min(jnp.where(masked == m, idxs, BIG), axis=1, keepdims=True)
    return m, ii


def _score_topk_body(q_ref, t_ref, s_ref, i_ref, p_ref):
    q = q_ref[...]
    t = t_ref[...]
    score = None
    for ax in range(3):
        qmn = jnp.minimum(jnp.minimum(q[:, ax:ax + 1], q[:, ax + 3:ax + 4]),
                          q[:, ax + 6:ax + 7])
        qmx = jnp.maximum(jnp.maximum(q[:, ax:ax + 1], q[:, ax + 3:ax + 4]),
                          q[:, ax + 6:ax + 7])
        tmn = jnp.minimum(jnp.minimum(t[ax:ax + 1, :], t[ax + 3:ax + 4, :]),
                          t[ax + 6:ax + 7, :])
        tmx = jnp.maximum(jnp.maximum(t[ax:ax + 1, :], t[ax + 3:ax + 4, :]),
                          t[ax + 6:ax + 7, :])
        ov = jnp.minimum(qmx, tmx) - jnp.maximum(qmn, tmn)
        score = ov if score is None else jnp.minimum(score, ov)

    iota = lax.broadcasted_iota(jnp.int32, score.shape, 1)
    masked = score
    ms, iis = [], []
    for k in range(K):
        m = jnp.max(masked, axis=1, keepdims=True)
        ii = jnp.min(jnp.where(masked == m, iota, BIG), axis=1, keepdims=True)
        ms.append(m)
        iis.append(ii)
        if k + 1 < K:
            masked = jnp.where(iota == ii, NEG, masked)
    s_ref[...] = jnp.concatenate(ms, axis=1)
    i_ref[...] = jnp.concatenate(iis, axis=1)
    p_ref[...] = jnp.concatenate(
        [(q[:, ax:ax + 1] + q[:, ax + 3:ax + 4] + q[:, ax + 6:ax + 7]) / 3.0
         for ax in range(3)], axis=1)


def _score_topk(q9p, t9T):
    return pl.pallas_call(
        _score_topk_body,
        grid=(PAD // RB,),
        in_specs=[
            pl.BlockSpec((RB, 9), lambda i: (i, 0)),
            pl.BlockSpec((9, PAD), lambda i: (0, 0)),
        ],
        out_specs=[
            pl.BlockSpec((RB, K), lambda i: (i, 0)),
            pl.BlockSpec((RB, K), lambda i: (i, 0)),
            pl.BlockSpec((RB, 3), lambda i: (i, 0)),
        ],
        out_shape=[
            jax.ShapeDtypeStruct((PAD, K), jnp.float32),
            jax.ShapeDtypeStruct((PAD, K), jnp.int32),
            jax.ShapeDtypeStruct((PAD, 3), jnp.float32),
        ],
    )(q9p, t9T)


NW = 32
RW = PAD // NW


def _bary_tail(idx_f, s_f, p_f, t9_f):
    mesh = plsc.VectorSubcoreMesh(core_axis_name="c", subcore_axis_name="s")
    cp = pltpu.CompilerParams()
    if "needs_layout_passes" in pltpu.CompilerParams.__dataclass_fields__:
        cp = dataclasses.replace(cp, needs_layout_passes=False)

    @functools.partial(
        pl.kernel, mesh=mesh, compiler_params=cp,
        out_type=[
            jax.ShapeDtypeStruct((PAD * K,), jnp.int32),
            jax.ShapeDtypeStruct((PAD * K,), jnp.float32),
            jax.ShapeDtypeStruct((PAD * K,), jnp.float32),
            jax.ShapeDtypeStruct((PAD * K,), jnp.float32),
        ],
        scratch_types=[
            pltpu.VMEM((RW * K,), jnp.int32),
            pltpu.VMEM((RW * K,), jnp.float32),
            pltpu.VMEM((RW * 3,), jnp.float32),
            pltpu.VMEM((PAD * 9,), jnp.float32),
            pltpu.VMEM((RW * K,), jnp.int32),
            pltpu.VMEM((RW * K,), jnp.float32),
            pltpu.VMEM((RW * K,), jnp.float32),
            pltpu.VMEM((RW * K,), jnp.float32),
        ],
    )
    def sc_k(idx_hbm, s_hbm, p_hbm, t9_hbm, f_hbm, u_hbm, v_hbm, w_hbm,
             idx_v, s_v, p_v, t9_v, f_v, u_v, v_v, w_v):
        wid = lax.axis_index("s") * 2 + lax.axis_index("c")
        base = wid * (RW * K)
        pbase = wid * (RW * 3)
        pltpu.sync_copy(idx_hbm.at[pl.ds(base, RW * K)], idx_v)
        pltpu.sync_copy(s_hbm.at[pl.ds(base, RW * K)], s_v)
        pltpu.sync_copy(p_hbm.at[pl.ds(pbase, RW * 3)], p_v)
        pltpu.sync_copy(t9_hbm, t9_v)

        zero16 = jnp.zeros((16,), jnp.int32)

        def row(r, carry):
            r3 = r * 3
            px = plsc.load_gather(p_v, [zero16 + r3])
            py = plsc.load_gather(p_v, [zero16 + (r3 + 1)])
            pz = plsc.load_gather(p_v, [zero16 + (r3 + 2)])
            for kk in range(K // 16):
                off = r * K + kk * 16
                sl = pl.ds(off, 16)
                idx16 = idx_v[sl]
                s16 = s_v[sl]
                b9 = idx16 * 9
                g = [plsc.load_gather(t9_v, [b9 + j]) for j in range(9)]
                axx, ayy, azz, bxx, byy, bzz, cxx, cyy, czz = g
                v0x, v0y, v0z = bxx - axx, byy - ayy, bzz - azz
                v1x, v1y, v1z = cxx - axx, cyy - ayy, czz - azz
                v2x, v2y, v2z = px - axx, py - ayy, pz - azz
                d00 = v0x * v0x + v0y * v0y + v0z * v0z
                d01 = v0x * v1x + v0y * v1y + v0z * v1z
                d11 = v1x * v1x + v1y * v1y + v1z * v1z
                d20 = v2x * v0x + v2y * v0y + v2z * v0z
                d21 = v2x * v1x + v2y * v1y + v2z * v1z
                denom = d00 * d11 - d01 * d01 + EPS
                vv = (d11 * d20 - d01 * d21) / denom
                ww = (d00 * d21 - d01 * d20) / denom
                uu = 1.0 - vv - ww
                valid = s16 > 0.0
                zf = jnp.zeros((16,), jnp.float32)
                f_v[sl] = jnp.where(valid, idx16, -1)
                u_v[sl] = jnp.where(valid, uu, zf)
                v_v[sl] = jnp.where(valid, vv, zf)
                w_v[sl] = jnp.where(valid, ww, zf)
            return carry

        lax.fori_loop(0, RW, row, 0)
        pltpu.sync_copy(f_v, f_hbm.at[pl.ds(base, RW * K)])
        pltpu.sync_copy(u_v, u_hbm.at[pl.ds(base, RW * K)])
        pltpu.sync_copy(v_v, v_hbm.at[pl.ds(base, RW * K)])
        pltpu.sync_copy(w_v, w_hbm.at[pl.ds(base, RW * K)])

    return sc_k(idx_f, s_f, p_f, t9_f)


def kernel(query_triangles, target_triangles):
    q9 = query_triangles.reshape(FQ, 9)
    t9 = target_triangles.reshape(FT, 9)
    q9p = jnp.pad(q9, ((0, PAD - FQ), (0, 0)))
    t9p = jnp.pad(t9, ((0, PAD - FT), (0, 0)), constant_values=TPAD)
    t9T = t9p.T

    top_s, top_i, p = _score_topk(q9p, t9T)

    f, u, v, w = _bary_tail(top_i.reshape(-1), top_s.reshape(-1),
                            p.reshape(-1), t9p.reshape(-1))

    faces = f.reshape(PAD, K)[:FQ]
    bcs = jnp.stack([u.reshape(PAD, K)[:FQ],
                     v.reshape(PAD, K)[:FQ],
                     w.reshape(PAD, K)[:FQ]], axis=-1)
    return (faces, bcs)
